# baseline (speedup 1.0000x reference)
import functools

import jax
import jax.numpy as jnp
from jax import lax
from jax.experimental import pallas as pl
from jax.experimental.pallas import tpu as pltpu
from jax.experimental.pallas import tpu_sc as plsc

_NC = 2
_NS = 16
_NW = _NC * _NS
_ROW = 128
_L = 16


def _node_proj(x_s, x_t, w1a, w1b):
    n = x_s.shape[0]
    blk = 5000
    grid = pl.cdiv(n, blk)

    def body(xs_ref, xt_ref, wa_ref, wb_ref, a_ref, b_ref):
        a_ref[...] = jnp.dot(xs_ref[...], wa_ref[...],
                             preferred_element_type=jnp.float32)
        b_ref[...] = jnp.dot(xt_ref[...], wb_ref[...],
                             preferred_element_type=jnp.float32)

    return pl.pallas_call(
        body,
        grid=(grid,),
        in_specs=[
            pl.BlockSpec((blk, x_s.shape[1]), lambda i: (i, 0)),
            pl.BlockSpec((blk, x_t.shape[1]), lambda i: (i, 0)),
            pl.BlockSpec(w1a.shape, lambda i: (0, 0)),
            pl.BlockSpec(w1b.shape, lambda i: (0, 0)),
        ],
        out_specs=[
            pl.BlockSpec((blk, w1a.shape[1]), lambda i: (i, 0)),
            pl.BlockSpec((blk, w1b.shape[1]), lambda i: (i, 0)),
        ],
        out_shape=[
            jax.ShapeDtypeStruct((n, w1a.shape[1]), jnp.float32),
            jax.ShapeDtypeStruct((n, w1b.shape[1]), jnp.float32),
        ],
    )(x_s, x_t, w1a, w1b)


def _u_proj(u, w1u, b1):
    def body(u_ref, w_ref, b_ref, o_ref):
        o_ref[...] = (jnp.dot(u_ref[...], w_ref[...],
                              preferred_element_type=jnp.float32)
                      + b_ref[...])

    return pl.pallas_call(
        body,
        out_shape=jax.ShapeDtypeStruct((u.shape[0], w1u.shape[1]),
                                       jnp.float32),
    )(u, w1u, b1.reshape(1, -1))


def _sc_edge_mlp(eidx, be, ea, a, bt, uq, wpack, rows):
    f_out = 5
    f_e = 10
    groups = _ROW // _L
    mesh = plsc.VectorSubcoreMesh(core_axis_name="c", subcore_axis_name="s")

    @functools.partial(
        pl.kernel,
        mesh=mesh,
        compiler_params=pltpu.CompilerParams(
            needs_layout_passes=False, use_tc_tiling_on_sc=False),
        out_type=jax.ShapeDtypeStruct((rows * _ROW * f_out,), jnp.float32),
        scratch_types=[
            pltpu.VMEM((2, _ROW), jnp.int32),
            pltpu.VMEM((2, _ROW), jnp.int32),
            pltpu.VMEM((2, _ROW), jnp.int32),
            pltpu.VMEM((2, _ROW * f_e), jnp.float32),
            pltpu.VMEM((2, _ROW, 8), jnp.float32),
            pltpu.VMEM((2, _ROW, 8), jnp.float32),
            pltpu.VMEM((uq.shape[0], f_out), jnp.float32),
            pltpu.VMEM((2, _ROW * f_out), jnp.float32),
            pltpu.VMEM((96,), jnp.float32),
            pltpu.SemaphoreType.DMA,
            pltpu.SemaphoreType.DMA,
            pltpu.SemaphoreType.DMA,
            pltpu.SemaphoreType.DMA,
            pltpu.SemaphoreType.DMA,
            pltpu.SemaphoreType.DMA,
            pltpu.SemaphoreType.DMA,
            pltpu.SemaphoreType.DMA,
            pltpu.SemaphoreType.DMA,
            pltpu.SemaphoreType.DMA,
            pltpu.SemaphoreType.DMA,
            pltpu.SemaphoreType.DMA,
            pltpu.SemaphoreType.DMA,
            pltpu.SemaphoreType.DMA,
        ],
    )
    def body(eidx_hbm, be_hbm, ea_hbm, a_hbm, bt_hbm, uq_hbm, w_hbm, out_hbm,
             sidx2, tidx2, bidx2, eav2, av2, bv2, uqv, ov2, wv,
             sem_s0, sem_t0, sem_b0, sem_e0, sem_a0, sem_bt0, sem_o0,
             sem_s1, sem_t1, sem_b1, sem_e1, sem_a1, sem_bt1, sem_o1):
        wid = lax.axis_index("s") * _NC + lax.axis_index("c")
        base = (rows * wid) // _NW
        end = (rows * (wid + 1)) // _NW

        pltpu.sync_copy(uq_hbm, uqv)
        pltpu.sync_copy(w_hbm, wv)
        wvecs = [wv[pl.ds(16 * i, 16)] for i in range(5)]
        wsc = [wvecs[i // 16][i % 16] for i in range(80)]
        w1c = [[wsc[f_out * k + j] for j in range(f_out)] for k in range(f_e)]
        w2 = [[wsc[50 + f_out * k + j] for j in range(f_out)]
              for k in range(f_out)]
        b2 = [wsc[75 + j] for j in range(f_out)]

        n_e = rows * _ROW
        slots = [
            (sidx2.at[0], tidx2.at[0], bidx2.at[0], eav2.at[0], av2.at[0],
             bv2.at[0], ov2.at[0], sem_s0, sem_t0, sem_b0, sem_e0, sem_a0,
             sem_bt0, sem_o0),
            (sidx2.at[1], tidx2.at[1], bidx2.at[1], eav2.at[1], av2.at[1],
             bv2.at[1], ov2.at[1], sem_s1, sem_t1, sem_b1, sem_e1, sem_a1,
             sem_bt1, sem_o1),
        ]

        def issue_lin(row, p):
            sidx, tidx, bidx, eav = slots[p][0], slots[p][1], slots[p][2], slots[p][3]
            sem_s, sem_t, sem_b, sem_e = slots[p][7], slots[p][8], slots[p][9], slots[p][10]
            e0 = row * _ROW
            ds = pltpu.async_copy(eidx_hbm.at[pl.ds(e0, _ROW)], sidx, sem_s)
            dt = pltpu.async_copy(eidx_hbm.at[pl.ds(n_e + e0, _ROW)], tidx,
                                  sem_t)
            db = pltpu.async_copy(be_hbm.at[pl.ds(e0, _ROW)], bidx, sem_b)
            de = pltpu.async_copy(ea_hbm.at[pl.ds(e0 * f_e, _ROW * f_e)], eav,
                                  sem_e)
            return ds, dt, db, de

        def issue_gath(p):
            sidx, tidx, av, bv = slots[p][0], slots[p][1], slots[p][4], slots[p][5]
            sem_a, sem_bt = slots[p][11], slots[p][12]
            ga = pltpu.async_copy(a_hbm.at[sidx], av, sem_a)
            gb = pltpu.async_copy(bt_hbm.at[tidx], bv, sem_bt)
            return ga, gb

        def compute(row, p):
            bidx, eav, av, bv, ov = (slots[p][2], slots[p][3], slots[p][4],
                                     slots[p][5], slots[p][6])
            sem_o = slots[p][13]
            for g in range(groups):
                ids = jnp.arange(_L, dtype=jnp.int32) + (_L * g)
                iota_fe = jnp.arange(_L, dtype=jnp.int32) * f_e
                bvec = bidx[pl.ds(_L * g, _L)]
                e_k = [plsc.load_gather(eav, [iota_fe + (_L * f_e * g + k)])
                       for k in range(f_e)]
                h = []
                for j in range(f_out):
                    jv = jnp.full((_L,), j, jnp.int32)
                    aj = plsc.load_gather(av, [ids, jv])
                    bj = plsc.load_gather(bv, [ids, jv])
                    uj = plsc.load_gather(uqv, [bvec, jv])
                    x = (aj + bj) + uj
                    for k in range(f_e):
                        x = x + e_k[k] * w1c[k][j]
                    h.append(x)
                h = [jnp.maximum(x, 0.1 * x) for x in h]
                iota_fo = jnp.arange(_L, dtype=jnp.int32) * f_out
                for j in range(f_out):
                    y = h[0] * w2[0][j]
                    for k in range(1, f_out):
                        y = y + h[k] * w2[k][j]
                    y = y + b2[j]
                    plsc.store_scatter(
                        ov, [iota_fo + (_L * f_out * g + j)], y)
            return pltpu.async_copy(
                ov, out_hbm.at[pl.ds(row * _ROW * f_out, _ROW * f_out)],
                sem_o)

        def pair_body(k, carry):
            ra = base + 2 * k
            rb = ra + 1
            la = issue_lin(ra, 0)
            lb = issue_lin(rb, 1)
            la[0].wait()
            la[1].wait()
            gaa = issue_gath(0)
            lb[0].wait()
            lb[1].wait()
            gab = issue_gath(1)
            la[2].wait()
            la[3].wait()
            gaa[0].wait()
            gaa[1].wait()
            oa = compute(ra, 0)
            lb[2].wait()
            lb[3].wait()
            gab[0].wait()
            gab[1].wait()
            ob = compute(rb, 1)
            oa.wait()
            ob.wait()
            return carry

        cnt = end - base
        lax.fori_loop(0, cnt // 2, pair_body, 0)

        @pl.when(cnt % 2 == 1)
        def _tail():
            r = end - 1
            lt = issue_lin(r, 0)
            lt[0].wait()
            lt[1].wait()
            gt = issue_gath(0)
            lt[2].wait()
            lt[3].wait()
            gt[0].wait()
            gt[1].wait()
            compute(r, 0).wait()

    return body(eidx, be, ea, a, bt, uq, wpack)


def kernel(x_s, x_t, edge_index, edge_attr, u, batch_e, W1, b1, W2, b2):
    e = edge_index.shape[1]
    f_xs = x_s.shape[1]
    f_xt = x_t.shape[1]
    f_e = edge_attr.shape[1]

    w1a = jnp.pad(W1[:f_xs], ((0, 0), (0, 3)))
    w1b = jnp.pad(W1[f_xs:f_xs + f_xt], ((0, 0), (0, 3)))
    w1c = W1[f_xs + f_xt:f_xs + f_xt + f_e]
    w1u = W1[f_xs + f_xt + f_e:]

    a, bt = _node_proj(x_s, x_t, w1a, w1b)
    uq = _u_proj(u, w1u, b1)

    rows = e // _ROW
    wpack = jnp.concatenate(
        [w1c.reshape(-1), W2.reshape(-1), b2,
         jnp.zeros((16,), jnp.float32)])

    out = _sc_edge_mlp(edge_index.reshape(-1), batch_e,
                       edge_attr.reshape(-1), a, bt, uq, wpack, rows)
    return out.reshape(e, 5)


# speedup vs baseline: 1.0093x; 1.0093x over previous
import functools

import jax
import jax.numpy as jnp
from jax import lax
from jax.experimental import pallas as pl
from jax.experimental.pallas import tpu as pltpu
from jax.experimental.pallas import tpu_sc as plsc

_NC = 2
_NS = 16
_NW = _NC * _NS
_ROW = 128
_L = 16


def _node_proj(x_s, x_t, w1a, w1b):
    n = x_s.shape[0]
    blk = 5000
    grid = pl.cdiv(n, blk)

    def body(xs_ref, xt_ref, wa_ref, wb_ref, a_ref, b_ref):
        a_ref[...] = jnp.dot(xs_ref[...], wa_ref[...],
                             preferred_element_type=jnp.float32)
        b_ref[...] = jnp.dot(xt_ref[...], wb_ref[...],
                             preferred_element_type=jnp.float32)

    return pl.pallas_call(
        body,
        grid=(grid,),
        in_specs=[
            pl.BlockSpec((blk, x_s.shape[1]), lambda i: (i, 0)),
            pl.BlockSpec((blk, x_t.shape[1]), lambda i: (i, 0)),
            pl.BlockSpec(w1a.shape, lambda i: (0, 0)),
            pl.BlockSpec(w1b.shape, lambda i: (0, 0)),
        ],
        out_specs=[
            pl.BlockSpec((blk, w1a.shape[1]), lambda i: (i, 0)),
            pl.BlockSpec((blk, w1b.shape[1]), lambda i: (i, 0)),
        ],
        out_shape=[
            jax.ShapeDtypeStruct((n, w1a.shape[1]), jnp.float32),
            jax.ShapeDtypeStruct((n, w1b.shape[1]), jnp.float32),
        ],
    )(x_s, x_t, w1a, w1b)


def _u_proj(u, w1u, b1):
    def body(u_ref, w_ref, b_ref, o_ref):
        o_ref[...] = (jnp.dot(u_ref[...], w_ref[...],
                              preferred_element_type=jnp.float32)
                      + b_ref[...])

    return pl.pallas_call(
        body,
        out_shape=jax.ShapeDtypeStruct((u.shape[0], w1u.shape[1]),
                                       jnp.float32),
    )(u, w1u, b1.reshape(1, -1))


def _sc_edge_mlp(eidx, be, ea, a, bt, uq, wpack, rows):
    f_out = 5
    f_e = 10
    groups = _ROW // _L
    mesh = plsc.VectorSubcoreMesh(core_axis_name="c", subcore_axis_name="s")

    @functools.partial(
        pl.kernel,
        mesh=mesh,
        compiler_params=pltpu.CompilerParams(
            needs_layout_passes=False, use_tc_tiling_on_sc=False),
        out_type=jax.ShapeDtypeStruct((rows * _ROW * f_out,), jnp.float32),
        scratch_types=[
            pltpu.VMEM((_ROW,), jnp.int32),
            pltpu.VMEM((_ROW,), jnp.int32),
            pltpu.VMEM((_ROW,), jnp.int32),
            pltpu.VMEM((_ROW * f_e,), jnp.float32),
            pltpu.VMEM((_ROW, 8), jnp.float32),
            pltpu.VMEM((_ROW, 8), jnp.float32),
            pltpu.VMEM((uq.shape[0], f_out), jnp.float32),
            pltpu.VMEM((_ROW * f_out,), jnp.float32),
            pltpu.VMEM((96,), jnp.float32),
            pltpu.SemaphoreType.DMA,
            pltpu.SemaphoreType.DMA,
            pltpu.SemaphoreType.DMA,
            pltpu.SemaphoreType.DMA,
            pltpu.SemaphoreType.DMA,
            pltpu.SemaphoreType.DMA,
        ],
    )
    def body(eidx_hbm, be_hbm, ea_hbm, a_hbm, bt_hbm, uq_hbm, w_hbm, out_hbm,
             sidx, tidx, bidx, eav, av, bv, uqv, ov, wv,
             sem_s0, sem_t0, sem_b0, sem_e0, sem_a0, sem_bt0):
        wid = lax.axis_index("s") * _NC + lax.axis_index("c")
        base = (rows * wid) // _NW
        end = (rows * (wid + 1)) // _NW

        pltpu.sync_copy(uq_hbm, uqv)
        pltpu.sync_copy(w_hbm, wv)
        wvecs = [wv[pl.ds(16 * i, 16)] for i in range(5)]
        wsc = [wvecs[i // 16][i % 16] for i in range(80)]
        w1c = [[wsc[f_out * k + j] for j in range(f_out)] for k in range(f_e)]
        w2 = [[wsc[50 + f_out * k + j] for j in range(f_out)]
              for k in range(f_out)]
        b2 = [wsc[75 + j] for j in range(f_out)]

        n_e = rows * _ROW

        def row_body(r, carry):
            e0 = (base + r) * _ROW
            ds = pltpu.async_copy(eidx_hbm.at[pl.ds(e0, _ROW)], sidx, sem_s0)
            dt = pltpu.async_copy(eidx_hbm.at[pl.ds(n_e + e0, _ROW)], tidx,
                                  sem_t0)
            db = pltpu.async_copy(be_hbm.at[pl.ds(e0, _ROW)], bidx, sem_b0)
            de = pltpu.async_copy(ea_hbm.at[pl.ds(e0 * f_e, _ROW * f_e)], eav,
                                  sem_e0)
            ds.wait()
            dt.wait()
            ga = pltpu.async_copy(a_hbm.at[sidx], av, sem_a0)
            gb = pltpu.async_copy(bt_hbm.at[tidx], bv, sem_bt0)
            db.wait()
            de.wait()
            ga.wait()
            gb.wait()

            for g in range(groups):
                ids = jnp.arange(_L, dtype=jnp.int32) + (_L * g)
                iota_fe = jnp.arange(_L, dtype=jnp.int32) * f_e
                bvec = bidx[pl.ds(_L * g, _L)]
                e_k = [plsc.load_gather(eav, [iota_fe + (_L * f_e * g + k)])
                       for k in range(f_e)]
                h = []
                for j in range(f_out):
                    jv = jnp.full((_L,), j, jnp.int32)
                    aj = plsc.load_gather(av, [ids, jv])
                    bj = plsc.load_gather(bv, [ids, jv])
                    uj = plsc.load_gather(uqv, [bvec, jv])
                    x = (aj + bj) + uj
                    for k in range(f_e):
                        x = x + e_k[k] * w1c[k][j]
                    h.append(x)
                h = [jnp.maximum(x, 0.1 * x) for x in h]
                iota_fo = jnp.arange(_L, dtype=jnp.int32) * f_out
                for j in range(f_out):
                    y = h[0] * w2[0][j]
                    for k in range(1, f_out):
                        y = y + h[k] * w2[k][j]
                    y = y + b2[j]
                    plsc.store_scatter(
                        ov, [iota_fo + (_L * f_out * g + j)], y)

            pltpu.sync_copy(ov, out_hbm.at[pl.ds(e0 * f_out, _ROW * f_out)])
            return carry

        lax.fori_loop(0, end - base, row_body, 0)

    return body(eidx, be, ea, a, bt, uq, wpack)


def kernel(x_s, x_t, edge_index, edge_attr, u, batch_e, W1, b1, W2, b2):
    e = edge_index.shape[1]
    f_xs = x_s.shape[1]
    f_xt = x_t.shape[1]
    f_e = edge_attr.shape[1]

    w1a = jnp.pad(W1[:f_xs], ((0, 0), (0, 3)))
    w1b = jnp.pad(W1[f_xs:f_xs + f_xt], ((0, 0), (0, 3)))
    w1c = W1[f_xs + f_xt:f_xs + f_xt + f_e]
    w1u = W1[f_xs + f_xt + f_e:]

    a, bt = _node_proj(x_s, x_t, w1a, w1b)
    uq = _u_proj(u, w1u, b1)

    rows = e // _ROW
    wpack = jnp.concatenate(
        [w1c.reshape(-1), W2.reshape(-1), b2,
         jnp.zeros((16,), jnp.float32)])

    out = _sc_edge_mlp(edge_index.reshape(-1), batch_e,
                       edge_attr.reshape(-1), a, bt, uq, wpack, rows)
    return out.reshape(e, 5)
